# Initial kernel scaffold; baseline (speedup 1.0000x reference)
import jax
import jax.numpy as jnp
from jax.experimental import pallas as pl


def kernel(x, edge_index, batch, W1, b1, W2, b2):
    raise NotImplementedError("write your pallas kernel here")


# speedup vs baseline: 24.0019x; 24.0019x over previous
import functools

import jax
import jax.numpy as jnp
from jax import lax
from jax.experimental import pallas as pl
from jax.experimental.pallas import tpu as pltpu
from jax.experimental.pallas import tpu_sc as plsc

N = 100000
E = 6400000
G = 128
D_IN = 5
D_HID = 30
D_OUT = 8
DP = 16

NC = 2
NS = 16
NW = NC * NS
EPW = E // NW
CH = 128
NFULL = EPW // CH
TAIL = EPW - NFULL * CH

RPT = N // NS
DEG_CHUNK = 5000
DEG_NCHUNK = N // DEG_CHUNK


def _sc_mesh():
    return plsc.VectorSubcoreMesh(
        core_axis_name="c", subcore_axis_name="s", num_cores=NC, num_subcores=NS
    )


def _sc_degree(dst, zeros_n8, ones_ch8):
    @functools.partial(
        pl.kernel,
        out_type=jax.ShapeDtypeStruct((NC * N, DP), jnp.float32),
        mesh=_sc_mesh(),
        compiler_params=pltpu.CompilerParams(use_tc_tiling_on_sc=False),
        scratch_types=[
            pltpu.VMEM_SHARED((N, DP), jnp.float32),
            pltpu.VMEM((CH,), jnp.int32),
            pltpu.VMEM((TAIL,), jnp.int32),
            pltpu.VMEM((CH, DP), jnp.float32),
            pltpu.SemaphoreType.DMA,
        ],
    )
    def body(dst_hbm, zeros_hbm, ones_hbm, out_hbm, deg_sh, idx_v, idxt_v, ones_v, sem):
        c = lax.axis_index("c")
        s = lax.axis_index("s")
        w = c * NS + s
        pltpu.sync_copy(zeros_hbm.at[pl.ds(s * DEG_CHUNK, DEG_CHUNK)],
                        deg_sh.at[pl.ds(s * DEG_CHUNK, DEG_CHUNK)])

        @pl.when(s < DEG_NCHUNK - NS)
        def _():
            pltpu.sync_copy(zeros_hbm.at[pl.ds((NS + s) * DEG_CHUNK, DEG_CHUNK)],
                            deg_sh.at[pl.ds((NS + s) * DEG_CHUNK, DEG_CHUNK)])

        pltpu.sync_copy(ones_hbm, ones_v)
        plsc.subcore_barrier()

        base = w * EPW

        def step(j, carry):
            pltpu.sync_copy(dst_hbm.at[pl.ds(base + j * CH, CH)], idx_v)
            pltpu.sync_copy(ones_v, deg_sh.at[idx_v], add=True)
            return carry

        lax.fori_loop(0, NFULL, step, 0)
        pltpu.sync_copy(dst_hbm.at[pl.ds(base + NFULL * CH, TAIL)], idxt_v)
        pltpu.sync_copy(ones_v.at[pl.ds(0, TAIL)], deg_sh.at[idxt_v], add=True)

        plsc.subcore_barrier()
        pltpu.sync_copy(deg_sh.at[pl.ds(s * DEG_CHUNK, DEG_CHUNK)],
                        out_hbm.at[pl.ds(c * N + s * DEG_CHUNK, DEG_CHUNK)])

        @pl.when(s < DEG_NCHUNK - NS)
        def _():
            pltpu.sync_copy(deg_sh.at[pl.ds((NS + s) * DEG_CHUNK, DEG_CHUNK)],
                            out_hbm.at[pl.ds(c * N + (NS + s) * DEG_CHUNK, DEG_CHUNK)])

    return body(dst, zeros_n8, ones_ch8)


def _sc_aggregate(table, src, dst, zeros_n8):
    @functools.partial(
        pl.kernel,
        out_type=jax.ShapeDtypeStruct((NC * N, DP), jnp.float32),
        mesh=_sc_mesh(),
        compiler_params=pltpu.CompilerParams(use_tc_tiling_on_sc=False),
        scratch_types=[
            pltpu.VMEM_SHARED((N, DP), jnp.float32),
            pltpu.VMEM((CH,), jnp.int32),
            pltpu.VMEM((CH,), jnp.int32),
            pltpu.VMEM((TAIL,), jnp.int32),
            pltpu.VMEM((TAIL,), jnp.int32),
            pltpu.VMEM((CH, DP), jnp.float32),
            pltpu.SemaphoreType.DMA,
            pltpu.SemaphoreType.DMA,
            pltpu.SemaphoreType.DMA,
        ],
    )
    def body(tab_hbm, src_hbm, dst_hbm, zeros_hbm, out_hbm,
             agg_sh, sidx, didx, sidxt, didxt, rows_v, sem_a, sem_b, sem_g):
        c = lax.axis_index("c")
        s = lax.axis_index("s")
        w = c * NS + s
        pltpu.sync_copy(zeros_hbm.at[pl.ds(s * DEG_CHUNK, DEG_CHUNK)],
                        agg_sh.at[pl.ds(s * DEG_CHUNK, DEG_CHUNK)])

        @pl.when(s < DEG_NCHUNK - NS)
        def _():
            pltpu.sync_copy(zeros_hbm.at[pl.ds((NS + s) * DEG_CHUNK, DEG_CHUNK)],
                            agg_sh.at[pl.ds((NS + s) * DEG_CHUNK, DEG_CHUNK)])

        plsc.subcore_barrier()

        base = w * EPW

        def step(j, carry):
            off = base + j * CH
            cp_s = pltpu.async_copy(src_hbm.at[pl.ds(off, CH)], sidx, sem_a)
            cp_d = pltpu.async_copy(dst_hbm.at[pl.ds(off, CH)], didx, sem_b)
            cp_s.wait()
            pltpu.async_copy(tab_hbm.at[sidx], rows_v, sem_g).wait()
            cp_d.wait()
            pltpu.sync_copy(rows_v, agg_sh.at[didx], add=True)
            return carry

        lax.fori_loop(0, NFULL, step, 0)
        off = base + NFULL * CH
        pltpu.sync_copy(src_hbm.at[pl.ds(off, TAIL)], sidxt)
        pltpu.sync_copy(dst_hbm.at[pl.ds(off, TAIL)], didxt)
        pltpu.async_copy(tab_hbm.at[sidxt], rows_v.at[pl.ds(0, TAIL)], sem_g).wait()
        pltpu.sync_copy(rows_v.at[pl.ds(0, TAIL)], agg_sh.at[didxt], add=True)

        plsc.subcore_barrier()
        pltpu.sync_copy(agg_sh.at[pl.ds(s * DEG_CHUNK, DEG_CHUNK)],
                        out_hbm.at[pl.ds(c * N + s * DEG_CHUNK, DEG_CHUNK)])

        @pl.when(s < DEG_NCHUNK - NS)
        def _():
            pltpu.sync_copy(agg_sh.at[pl.ds((NS + s) * DEG_CHUNK, DEG_CHUNK)],
                            out_hbm.at[pl.ds(c * N + (NS + s) * DEG_CHUNK, DEG_CHUNK)])

    return body(table, src, dst, zeros_n8)


BN = 2000
NBLK = N // BN


def _tc_prep(deg2, x):
    def body(deg_ref, x_ref, dinv_ref, u_ref):
        deg = deg_ref[0, :, 0] + deg_ref[1, :, 0] + 1.0
        dinv = lax.rsqrt(deg)
        dinv_ref[...] = dinv[:, None]
        ux = x_ref[...] * dinv[:, None]
        pad = jnp.zeros((BN, DP - D_IN), jnp.float32)
        u_ref[...] = jnp.concatenate([ux, pad], axis=1)

    return pl.pallas_call(
        body,
        grid=(NBLK,),
        in_specs=[
            pl.BlockSpec((NC, BN, DP), lambda i: (0, i, 0)),
            pl.BlockSpec((BN, D_IN), lambda i: (i, 0)),
        ],
        out_specs=[
            pl.BlockSpec((BN, 1), lambda i: (i, 0)),
            pl.BlockSpec((BN, DP), lambda i: (i, 0)),
        ],
        out_shape=[
            jax.ShapeDtypeStruct((N, 1), jnp.float32),
            jax.ShapeDtypeStruct((N, DP), jnp.float32),
        ],
    )(deg2, x)


def _tc_mid(agg1, u, dinv, W1, b1, W2):
    def body(a_ref, u_ref, dinv_ref, w1_ref, b1_ref, w2_ref, q_ref):
        dinv = dinv_ref[...]
        z8 = dinv * (a_ref[0] + a_ref[1] + u_ref[...])
        z = z8[:, :D_IN]
        h = jnp.dot(z, w1_ref[...], preferred_element_type=jnp.float32) + b1_ref[...][None, :]
        h = jnp.maximum(h, 0.0)
        q = jnp.dot(h, w2_ref[...], preferred_element_type=jnp.float32)
        qpad = jnp.concatenate([q, jnp.zeros((BN, DP - D_OUT), jnp.float32)], axis=1)
        q_ref[...] = dinv * qpad

    return pl.pallas_call(
        body,
        grid=(NBLK,),
        in_specs=[
            pl.BlockSpec((NC, BN, DP), lambda i: (0, i, 0)),
            pl.BlockSpec((BN, DP), lambda i: (i, 0)),
            pl.BlockSpec((BN, 1), lambda i: (i, 0)),
            pl.BlockSpec((D_IN, D_HID), lambda i: (0, 0)),
            pl.BlockSpec((D_HID,), lambda i: (0,)),
            pl.BlockSpec((D_HID, D_OUT), lambda i: (0, 0)),
        ],
        out_specs=pl.BlockSpec((BN, DP), lambda i: (i, 0)),
        out_shape=jax.ShapeDtypeStruct((N, DP), jnp.float32),
    )(agg1, u, dinv, W1, b1, W2)


def _tc_pool(agg2, q, dinv, batch, b2):
    def body(a_ref, q_ref, dinv_ref, batch_ref, b2_ref, out_ref, cnt_ref):
        i = pl.program_id(0)

        @pl.when(i == 0)
        def _():
            out_ref[...] = jnp.zeros((G, D_OUT), jnp.float32)
            cnt_ref[...] = jnp.zeros((G, 1), jnp.float32)

        dinv = dinv_ref[...]
        o2 = dinv * (a_ref[0] + a_ref[1] + q_ref[...])
        gids = lax.broadcasted_iota(jnp.int32, (G, BN), 0)
        P = (batch_ref[...][:, 0][None, :] == gids).astype(jnp.float32)
        out_ref[...] += jnp.dot(P, o2[:, :D_OUT], preferred_element_type=jnp.float32)
        cnt_ref[...] += jnp.sum(P, axis=1, keepdims=True)

        @pl.when(i == NBLK - 1)
        def _():
            cnt = cnt_ref[...]
            out_ref[...] = (out_ref[...] + cnt * b2_ref[...][None, :]) / jnp.maximum(cnt, 1.0)

    return pl.pallas_call(
        body,
        grid=(NBLK,),
        in_specs=[
            pl.BlockSpec((NC, BN, DP), lambda i: (0, i, 0)),
            pl.BlockSpec((BN, DP), lambda i: (i, 0)),
            pl.BlockSpec((BN, 1), lambda i: (i, 0)),
            pl.BlockSpec((BN, 1), lambda i: (i, 0)),
            pl.BlockSpec((D_OUT,), lambda i: (0,)),
        ],
        out_specs=pl.BlockSpec((G, D_OUT), lambda i: (0, 0)),
        out_shape=jax.ShapeDtypeStruct((G, D_OUT), jnp.float32),
        scratch_shapes=[pltpu.VMEM((G, 1), jnp.float32)],
    )(agg2, q, dinv, batch, b2)


def kernel(x, edge_index, batch, W1, b1, W2, b2):
    src = edge_index[0].astype(jnp.int32)
    dst = edge_index[1].astype(jnp.int32)
    batch = batch.astype(jnp.int32)
    x = x.astype(jnp.float32)

    zeros_n8 = jnp.zeros((N, DP), jnp.float32)
    ones_ch8 = jnp.ones((CH, DP), jnp.float32)

    deg2 = _sc_degree(dst, zeros_n8, ones_ch8).reshape(NC, N, DP)
    dinv, u = _tc_prep(deg2, x)
    agg1 = _sc_aggregate(u, src, dst, zeros_n8).reshape(NC, N, DP)
    q = _tc_mid(agg1, u, dinv, W1, b1, W2)
    agg2 = _sc_aggregate(q, src, dst, zeros_n8).reshape(NC, N, DP)
    return _tc_pool(agg2, q, dinv, batch.reshape(N, 1), b2)
